# baseline (speedup 1.0000x reference)
import functools

import jax
import jax.numpy as jnp
from jax import lax
from jax.experimental import pallas as pl
from jax.experimental.pallas import tpu as pltpu
from jax.experimental.pallas import tpu_sc as plsc

_NC = 2
_NS = 16
_NW = _NC * _NS
_LANES = 16

_N = 8192
_D = 1024
_L = 2048
_RPW = _N // _NW
_C = 16
_NCHUNK = _RPW // _C
_EPS = 1e-12


def _emb_body(x_hbm, seg_hbm, tok_hbm, pos_hbm, segtab_hbm, gamma_hbm,
              beta_hbm, out_hbm, tokidx_v, segidx_v, posidx_v, e_v, g_v, b_v,
              sem0, sem1):
    wid = lax.axis_index("s") * _NC + lax.axis_index("c")
    base = wid * _RPW
    l_base = lax.rem(base, _L)

    pltpu.sync_copy(x_hbm.at[pl.ds(base, _RPW)], tokidx_v)
    pltpu.sync_copy(seg_hbm.at[pl.ds(base, _RPW)], segidx_v)
    pltpu.sync_copy(gamma_hbm, g_v)
    pltpu.sync_copy(beta_hbm, b_v)

    lane = lax.iota(jnp.int32, _LANES)
    zero = jnp.zeros((_LANES,), jnp.float32)

    @pl.loop(0, _NCHUNK)
    def _fill(c):
        posidx_v[pl.ds(c * _C, _C)] = l_base + c * _C + lane

    @pl.loop(0, _NCHUNK)
    def _chunk(c):
        r0 = c * _C

        pltpu.async_copy(tok_hbm.at[tokidx_v.at[pl.ds(r0, _C)]],
                         e_v, sem0).wait()
        pltpu.async_copy(pos_hbm.at[posidx_v.at[pl.ds(r0, _C)]],
                         e_v, sem0, add=True).wait()
        pltpu.async_copy(segtab_hbm.at[segidx_v.at[pl.ds(r0, _C)]],
                         e_v, sem1, add=True).wait()

        def p1(d, carry):
            s, s2, col = carry
            v = plsc.load_gather(e_v, [lane, col])
            return s + v, s2 + v * v, (col + 1) & (_D - 1)

        s, s2, _ = lax.fori_loop(0, _D, p1, (zero, zero, lane), unroll=8)

        u = s * (1.0 / _D)
        var = s2 * (1.0 / _D) - u * u
        t = var + _EPS
        y = lax.bitcast_convert_type(
            jnp.int32(0x5F3759DF) - (lax.bitcast_convert_type(t, jnp.int32) >> 1),
            jnp.float32)
        y = y * (1.5 - 0.5 * t * y * y)
        y = y * (1.5 - 0.5 * t * y * y)
        y = y * (1.5 - 0.5 * t * y * y)
        rstd = y

        def p2(d, col):
            v = plsc.load_gather(e_v, [lane, col])
            g = plsc.load_gather(g_v, [col])
            b = plsc.load_gather(b_v, [col])
            o = (v - u) * rstd * g + b
            plsc.store_scatter(e_v, [lane, col], o)
            return (col + 1) & (_D - 1)

        lax.fori_loop(0, _D, p2, lane, unroll=8)

        pltpu.sync_copy(e_v, out_hbm.at[pl.ds(base + r0, _C)])


@jax.jit
def _emb(x_flat, seg_flat, tok_table, pos_table, seg_table, gamma, beta):
    mesh = plsc.VectorSubcoreMesh(core_axis_name="c", subcore_axis_name="s",
                                  num_cores=_NC, num_subcores=_NS)
    run = pl.kernel(
        _emb_body,
        out_type=jax.ShapeDtypeStruct((_N, _D), jnp.float32),
        mesh=mesh,
        scratch_types=[
            pltpu.VMEM((_RPW,), jnp.int32),
            pltpu.VMEM((_RPW,), jnp.int32),
            pltpu.VMEM((_RPW,), jnp.int32),
            pltpu.VMEM((_C, _D), jnp.float32),
            pltpu.VMEM((_D,), jnp.float32),
            pltpu.VMEM((_D,), jnp.float32),
            pltpu.SemaphoreType.DMA,
            pltpu.SemaphoreType.DMA,
        ],
        compiler_params=pltpu.CompilerParams(use_tc_tiling_on_sc=False,
                                             needs_layout_passes=False),
    )
    return run(x_flat, seg_flat, tok_table, pos_table, seg_table, gamma, beta)


def kernel(x, seg, mixup, shuffle_idx, l, clone_ids, mixup_layer, simple_pad,
           no_grad_clone, tok_table, pos_table, seg_table, gamma, beta):
    B, L = x.shape
    out = _emb(x.reshape(-1), seg.reshape(-1), tok_table, pos_table,
               seg_table, gamma, beta)
    return out.reshape(B, L, tok_table.shape[1])


# speedup vs baseline: 1.0394x; 1.0394x over previous
import functools

import jax
import jax.numpy as jnp
from jax import lax
from jax.experimental import pallas as pl
from jax.experimental.pallas import tpu as pltpu
from jax.experimental.pallas import tpu_sc as plsc

_NC = 2
_NS = 16
_NW = _NC * _NS
_LANES = 16

_N = 8192
_D = 1024
_L = 2048
_RPW = _N // _NW
_C = 16
_NCHUNK = _RPW // _C
_NPAIR = _NCHUNK // 2
_EPS = 1e-12


def _ln_chunk(e_v, o_v, g_v, b_v, lane):
    zero = jnp.zeros((_LANES,), jnp.float32)

    def p1(d, carry):
        s, s2, col = carry
        v = plsc.load_gather(e_v, [lane, col])
        return s + v, s2 + v * v, (col + 1) & (_D - 1)

    s, s2, _ = lax.fori_loop(0, _D, p1, (zero, zero, lane), unroll=8)

    u = s * (1.0 / _D)
    var = s2 * (1.0 / _D) - u * u
    t = var + _EPS
    y = lax.bitcast_convert_type(
        jnp.int32(0x5F3759DF) - (lax.bitcast_convert_type(t, jnp.int32) >> 1),
        jnp.float32)
    y = y * (1.5 - 0.5 * t * y * y)
    y = y * (1.5 - 0.5 * t * y * y)
    y = y * (1.5 - 0.5 * t * y * y)
    rstd = y

    def p2(d, col):
        v = plsc.load_gather(e_v, [lane, col])
        g = plsc.load_gather(g_v, [col])
        b = plsc.load_gather(b_v, [col])
        o = (v - u) * rstd * g + b
        plsc.store_scatter(o_v, [lane, col], o)
        plsc.store_scatter(e_v, [lane, col], zero)
        return (col + 1) & (_D - 1)

    lax.fori_loop(0, _D, p2, lane, unroll=8)


def _emb_body(x_hbm, seg_hbm, zeros_hbm, tok_hbm, pos_hbm, segtab_hbm,
              gamma_hbm, beta_hbm, out_hbm, tokidx_v, segidx_v, posidx_v,
              e0, e1, o0, o1, g_v, b_v, semA, semB, semOA, semOB):
    wid = lax.axis_index("s") * _NC + lax.axis_index("c")
    base = wid * _RPW
    l_base = lax.rem(base, _L)

    lane = lax.iota(jnp.int32, _LANES)

    pltpu.async_copy(x_hbm.at[pl.ds(base, _RPW)], tokidx_v, semA)
    pltpu.async_copy(seg_hbm.at[pl.ds(base, _RPW)], segidx_v, semA)
    pltpu.async_copy(gamma_hbm, g_v, semA)
    pltpu.async_copy(beta_hbm, b_v, semA)
    pltpu.async_copy(zeros_hbm, e0, semB)
    pltpu.async_copy(zeros_hbm, e1, semB)
    pltpu.make_async_copy(x_hbm.at[pl.ds(base, _RPW)], tokidx_v, semA).wait()
    pltpu.make_async_copy(seg_hbm.at[pl.ds(base, _RPW)], segidx_v, semA).wait()
    pltpu.make_async_copy(gamma_hbm, g_v, semA).wait()
    pltpu.make_async_copy(beta_hbm, b_v, semA).wait()
    pltpu.make_async_copy(zeros_hbm, e0, semB).wait()
    pltpu.make_async_copy(zeros_hbm, e1, semB).wait()

    @pl.loop(0, _NCHUNK)
    def _fill(c):
        posidx_v[pl.ds(c * _C, _C)] = l_base + c * _C + lane

    def issue_adds(c, e_v, sem):
        r0 = c * _C
        pltpu.async_copy(tok_hbm.at[tokidx_v.at[pl.ds(r0, _C)]],
                         e_v, sem, add=True)
        pltpu.async_copy(pos_hbm.at[posidx_v.at[pl.ds(r0, _C)]],
                         e_v, sem, add=True)
        pltpu.async_copy(segtab_hbm.at[segidx_v.at[pl.ds(r0, _C)]],
                         e_v, sem, add=True)

    def drain_adds(e_v, sem):
        for _ in range(3):
            pltpu.make_async_copy(tok_hbm.at[tokidx_v.at[pl.ds(0, _C)]],
                                  e_v, sem).wait()

    def drain_out(c, o_v, sem):
        pltpu.make_async_copy(o_v, out_hbm.at[pl.ds(base + c * _C, _C)],
                              sem).wait()

    issue_adds(0, e0, semA)
    issue_adds(1, e1, semB)

    @pl.loop(0, _NPAIR)
    def _pair(i):
        cA = 2 * i
        cB = 2 * i + 1

        drain_adds(e0, semA)

        @pl.when(i > 0)
        def _():
            drain_out(cA, o0, semOA)

        _ln_chunk(e0, o0, g_v, b_v, lane)
        pltpu.async_copy(o0, out_hbm.at[pl.ds(base + cA * _C, _C)], semOA)

        @pl.when(i < _NPAIR - 1)
        def _():
            issue_adds(cA + 2, e0, semA)

        drain_adds(e1, semB)

        @pl.when(i > 0)
        def _():
            drain_out(cB, o1, semOB)

        _ln_chunk(e1, o1, g_v, b_v, lane)
        pltpu.async_copy(o1, out_hbm.at[pl.ds(base + cB * _C, _C)], semOB)

        @pl.when(i < _NPAIR - 1)
        def _():
            issue_adds(cB + 2, e1, semB)

    drain_out(_NCHUNK - 2, o0, semOA)
    drain_out(_NCHUNK - 1, o1, semOB)


@jax.jit
def _emb(x_flat, seg_flat, zeros, tok_table, pos_table, seg_table, gamma,
         beta):
    mesh = plsc.VectorSubcoreMesh(core_axis_name="c", subcore_axis_name="s",
                                  num_cores=_NC, num_subcores=_NS)
    run = pl.kernel(
        _emb_body,
        out_type=jax.ShapeDtypeStruct((_N, _D), jnp.float32),
        mesh=mesh,
        scratch_types=[
            pltpu.VMEM((_RPW,), jnp.int32),
            pltpu.VMEM((_RPW,), jnp.int32),
            pltpu.VMEM((_RPW,), jnp.int32),
            pltpu.VMEM((_C, _D), jnp.float32),
            pltpu.VMEM((_C, _D), jnp.float32),
            pltpu.VMEM((_C, _D), jnp.float32),
            pltpu.VMEM((_C, _D), jnp.float32),
            pltpu.VMEM((_D,), jnp.float32),
            pltpu.VMEM((_D,), jnp.float32),
            pltpu.SemaphoreType.DMA,
            pltpu.SemaphoreType.DMA,
            pltpu.SemaphoreType.DMA,
            pltpu.SemaphoreType.DMA,
        ],
        compiler_params=pltpu.CompilerParams(use_tc_tiling_on_sc=False,
                                             needs_layout_passes=False),
    )
    return run(x_flat, seg_flat, zeros, tok_table, pos_table, seg_table,
               gamma, beta)


def kernel(x, seg, mixup, shuffle_idx, l, clone_ids, mixup_layer, simple_pad,
           no_grad_clone, tok_table, pos_table, seg_table, gamma, beta):
    B, L = x.shape
    zeros = jnp.zeros((_C, _D), jnp.float32)
    out = _emb(x.reshape(-1), seg.reshape(-1), zeros, tok_table, pos_table,
               seg_table, gamma, beta)
    return out.reshape(B, L, tok_table.shape[1])


# speedup vs baseline: 1.8204x; 1.7514x over previous
import functools

import jax
import jax.numpy as jnp
from jax import lax
from jax.experimental import pallas as pl
from jax.experimental.pallas import tpu as pltpu
from jax.experimental.pallas import tpu_sc as plsc

_NC = 2
_NS = 16
_NW = _NC * _NS
_LANES = 16

_N = 8192
_D = 1024
_L = 2048
_J = _D // 128
_RPW = _N // _NW
_C = 16
_NCHUNK = _RPW // _C
_NPAIR = _NCHUNK // 2
_QIDX = _C * _J
_EPS = 1e-12


def _ln_chunk(e_v, o_v, g_v, b_v):
    zero = jnp.zeros((_LANES,), jnp.float32)

    def p1(k, carry):
        accs = carry
        j16 = (k >> 3) * _LANES
        off = (k & 7) * _LANES
        out = []
        for r in range(_C):
            v = e_v[j16 + r, pl.ds(off, _LANES)]
            out.append(accs[2 * r] + v)
            out.append(accs[2 * r + 1] + v * v)
        return tuple(out)

    accs = lax.fori_loop(0, _D // _LANES, p1, (zero,) * (2 * _C), unroll=2)

    rstds, urss = [], []
    for r in range(_C):
        s = jnp.sum(accs[2 * r])
        q = jnp.sum(accs[2 * r + 1])
        u = s * (1.0 / _D)
        t = q * (1.0 / _D) - u * u + _EPS
        y = lax.bitcast_convert_type(
            jnp.int32(0x5F3759DF)
            - (lax.bitcast_convert_type(t, jnp.int32) >> 1), jnp.float32)
        y = y * (1.5 - 0.5 * t * y * y)
        y = y * (1.5 - 0.5 * t * y * y)
        y = y * (1.5 - 0.5 * t * y * y)
        rstds.append(y)
        urss.append(u * y)

    def p2(kb, _):
        base = kb * 256
        j16b = (kb * 2) * _LANES
        gs = [g_v[pl.ds(base + j * _LANES, _LANES)] for j in range(16)]
        bs = [b_v[pl.ds(base + j * _LANES, _LANES)] for j in range(16)]
        for r in range(_C):
            rst = rstds[r]
            urs = urss[r]
            for j in range(16):
                erow = j16b + (j >> 3) * _LANES + r
                orow = ((r >> 3) * 64 + (r & 7)) + (kb * 2 + (j >> 3)) * 8
                sl = pl.ds((j & 7) * _LANES, _LANES)
                v = e_v[erow, sl]
                o_v[orow, sl] = (v * rst - urs) * gs[j] + bs[j]
                e_v[erow, sl] = zero
        return 0

    lax.fori_loop(0, _D // 256, p2, 0, unroll=1)


def _emb_body(x_hbm, seg_hbm, zeros_hbm, tok_hbm, pos_hbm, segtab_hbm,
              gamma_hbm, beta_hbm, out_hbm, tokidx_v, segidx_v, q0, q1,
              e0, e1, o0, o1, g_v, b_v, semA, semB, semOA, semOB):
    wid = lax.axis_index("s") * _NC + lax.axis_index("c")
    base = wid * _RPW
    l_base = lax.rem(base, _L)

    lane = lax.iota(jnp.int32, _LANES)

    xrow = base >> 7
    pltpu.async_copy(x_hbm.at[pl.ds(xrow, 2)], tokidx_v, semA)
    pltpu.async_copy(seg_hbm.at[pl.ds(xrow, 2)], segidx_v, semA)
    pltpu.async_copy(gamma_hbm, g_v, semA)
    pltpu.async_copy(beta_hbm, b_v, semA)
    pltpu.async_copy(zeros_hbm, e0, semB)
    pltpu.async_copy(zeros_hbm, e1, semB)
    pltpu.make_async_copy(x_hbm.at[pl.ds(xrow, 2)], tokidx_v, semA).wait()
    pltpu.make_async_copy(seg_hbm.at[pl.ds(xrow, 2)], segidx_v, semA).wait()
    pltpu.make_async_copy(gamma_hbm, g_v, semA).wait()
    pltpu.make_async_copy(beta_hbm, b_v, semA).wait()
    pltpu.make_async_copy(zeros_hbm, e0, semB).wait()
    pltpu.make_async_copy(zeros_hbm, e1, semB).wait()

    def issue_adds(c, q_v, e_v, sem):
        r0 = c * _C
        tv = tokidx_v[c >> 3, pl.ds((c & 7) * _C, _C)]
        tq = ((tv >> 3) << 6) | (tv & 7)
        pv = l_base + r0 + lane
        pq = ((pv >> 3) << 6) | (pv & 7)
        sv = segidx_v[c >> 3, pl.ds((c & 7) * _C, _C)]
        sq = sv << 3
        for j in range(_J):
            q_v[pl.ds(j * _LANES, _LANES)] = tq + j * 8
            q_v[pl.ds(_QIDX + j * _LANES, _LANES)] = pq + j * 8
            q_v[pl.ds(2 * _QIDX + j * _LANES, _LANES)] = sq + j
        pltpu.async_copy(tok_hbm.at[q_v.at[pl.ds(0, _QIDX)]],
                         e_v, sem, add=True)
        pltpu.async_copy(pos_hbm.at[q_v.at[pl.ds(_QIDX, _QIDX)]],
                         e_v, sem, add=True)
        pltpu.async_copy(segtab_hbm.at[q_v.at[pl.ds(2 * _QIDX, _QIDX)]],
                         e_v, sem, add=True)

    def drain_adds(e_v, sem):
        for _ in range(3):
            pltpu.make_async_copy(tok_hbm.at[q0.at[pl.ds(0, _QIDX)]],
                                  e_v, sem).wait()

    def out_slice(c):
        return out_hbm.at[pl.ds((base + c * _C) * _J, _QIDX)]

    def drain_out(c, o_v, sem):
        pltpu.make_async_copy(o_v, out_slice(c), sem).wait()

    issue_adds(0, q0, e0, semA)
    issue_adds(1, q1, e1, semB)

    @pl.loop(0, _NPAIR)
    def _pair(i):
        cA = 2 * i
        cB = 2 * i + 1

        drain_adds(e0, semA)

        @pl.when(i > 0)
        def _():
            drain_out(cA, o0, semOA)

        _ln_chunk(e0, o0, g_v, b_v)
        pltpu.async_copy(o0, out_slice(cA), semOA)

        @pl.when(i < _NPAIR - 1)
        def _():
            issue_adds(cA + 2, q0, e0, semA)

        drain_adds(e1, semB)

        @pl.when(i > 0)
        def _():
            drain_out(cB, o1, semOB)

        _ln_chunk(e1, o1, g_v, b_v)
        pltpu.async_copy(o1, out_slice(cB), semOB)

        @pl.when(i < _NPAIR - 1)
        def _():
            issue_adds(cB + 2, q1, e1, semB)

    drain_out(_NCHUNK - 2, o0, semOA)
    drain_out(_NCHUNK - 1, o1, semOB)


@jax.jit
def _emb(x2, seg2, zeros, tok_q, pos_q, seg_q, gamma, beta):
    mesh = plsc.VectorSubcoreMesh(core_axis_name="c", subcore_axis_name="s",
                                  num_cores=_NC, num_subcores=_NS)
    run = pl.kernel(
        _emb_body,
        out_type=jax.ShapeDtypeStruct((_N * _J, 128), jnp.float32),
        mesh=mesh,
        scratch_types=[
            pltpu.VMEM((2, 128), jnp.int32),
            pltpu.VMEM((2, 128), jnp.int32),
            pltpu.VMEM((3 * _QIDX,), jnp.int32),
            pltpu.VMEM((3 * _QIDX,), jnp.int32),
            pltpu.VMEM((_QIDX, 128), jnp.float32),
            pltpu.VMEM((_QIDX, 128), jnp.float32),
            pltpu.VMEM((_QIDX, 128), jnp.float32),
            pltpu.VMEM((_QIDX, 128), jnp.float32),
            pltpu.VMEM((_D,), jnp.float32),
            pltpu.VMEM((_D,), jnp.float32),
            pltpu.SemaphoreType.DMA,
            pltpu.SemaphoreType.DMA,
            pltpu.SemaphoreType.DMA,
            pltpu.SemaphoreType.DMA,
        ],
        compiler_params=pltpu.CompilerParams(use_tc_tiling_on_sc=False,
                                             needs_layout_passes=False),
    )
    return run(x2, seg2, zeros, tok_q, pos_q, seg_q, gamma, beta)


def _to_q(t):
    r = t.shape[0]
    return (t.reshape(r // 8, 8, 8, 128)
             .transpose(0, 2, 1, 3)
             .reshape(r * 8, 128))


def _from_q(o, B, L, D):
    return (o.reshape(B * L // 8, 8, 8, 128)
             .transpose(0, 2, 1, 3)
             .reshape(B, L, D))


def kernel(x, seg, mixup, shuffle_idx, l, clone_ids, mixup_layer, simple_pad,
           no_grad_clone, tok_table, pos_table, seg_table, gamma, beta):
    B, L = x.shape
    D = tok_table.shape[1]
    zeros = jnp.zeros((_QIDX, 128), jnp.float32)
    out = _emb(x.reshape(-1, 128), seg.reshape(-1, 128), zeros,
               _to_q(tok_table), _to_q(pos_table),
               seg_table.reshape(-1, 128), gamma, beta)
    return _from_q(out, B, L, D)


# speedup vs baseline: 4.5590x; 2.5044x over previous
import jax
import jax.numpy as jnp
from jax import lax
from jax.experimental import pallas as pl
from jax.experimental.pallas import tpu as pltpu
from jax.experimental.pallas import tpu_sc as plsc

_NC = 2
_NS = 16
_NW = _NC * _NS
_LANES = 16

_N = 8192
_D = 1024
_L = 2048
_J = _D // 128
_RPW = _N // _NW
_C = 16
_NCHUNK = _RPW // _C
_NPAIR = _NCHUNK // 2
_QIDX = _C * _J
_EPS = 1e-12


def _ln_chunk(e3_v, o_v, g_v, b_v, segbuf_v, acc_v, stat_v, sgs):
    zero = jnp.zeros((_LANES,), jnp.float32)
    e_v = e3_v.at[0]

    def p1(k, carry):
        accs = carry
        j16 = (k >> 3) * _LANES
        off = (k & 7) * _LANES
        out = []
        for r in range(_C):
            sl = pl.ds(off, _LANES)
            v = e_v[j16 + r, sl] + segbuf_v[sgs[r] + (k >> 3), sl]
            e_v[j16 + r, sl] = v
            out.append(accs[2 * r] + v)
            out.append(accs[2 * r + 1] + v * v)
        return tuple(out)

    accs = lax.fori_loop(0, _D // _LANES, p1, (zero,) * (2 * _C), unroll=1)
    for r in range(_C):
        acc_v[0, pl.ds(r * _LANES, _LANES)] = accs[2 * r]
        acc_v[1, pl.ds(r * _LANES, _LANES)] = accs[2 * r + 1]

    @plsc.parallel_loop(0, _C, unroll=2)
    def stats(r):
        s = jnp.sum(acc_v[0, pl.ds(r * _LANES, _LANES)])
        q = jnp.sum(acc_v[1, pl.ds(r * _LANES, _LANES)])
        u = s * (1.0 / _D)
        t = q * (1.0 / _D) - u * u + _EPS
        y = lax.bitcast_convert_type(
            jnp.int32(0x5F3759DF)
            - (lax.bitcast_convert_type(t, jnp.int32) >> 1), jnp.float32)
        y = y * (1.5 - 0.5 * t * y * y)
        y = y * (1.5 - 0.5 * t * y * y)
        y = y * (1.5 - 0.5 * t * y * y)
        stat_v[0, r] = y
        stat_v[1, r] = u * y

    @plsc.parallel_loop(0, _J)
    def p2(kb):
        gs = [g_v[pl.ds(kb * 128 + j * _LANES, _LANES)] for j in range(8)]
        bs = [b_v[pl.ds(kb * 128 + j * _LANES, _LANES)] for j in range(8)]

        @plsc.parallel_loop(0, _C, unroll=2)
        def p2r(r):
            rst = stat_v[0, r]
            urs = stat_v[1, r]
            erow = kb * _LANES + r
            orow = ((r >> 3) * 64 + (r & 7)) + kb * 8
            for j in range(8):
                sl = pl.ds(j * _LANES, _LANES)
                v = e_v[erow, sl]
                o_v[orow, sl] = (v * rst - urs) * gs[j] + bs[j]
                e_v[erow, sl] = zero


def _emb_body(x_hbm, seg_hbm, tok_hbm, pos_hbm, segtab_hbm,
              gamma_hbm, beta_hbm, out_hbm, tokidx_v, segidx_v, q0, q1,
              oidx_v, posidx_v, e0, e1, o0, o1, segbuf_v, g_v, b_v,
              acc_v, stat_v, semA, semB, semOA, semOB):
    wid = lax.axis_index("s") * _NC + lax.axis_index("c")
    base = wid * _RPW
    l_base = lax.rem(base, _L)

    lane = lax.iota(jnp.int32, _LANES)
    zerov = jnp.zeros((_LANES,), jnp.float32)

    xrow = base >> 7
    pltpu.async_copy(x_hbm.at[pl.ds(xrow, 2)], tokidx_v, semA)
    pltpu.async_copy(seg_hbm.at[pl.ds(xrow, 2)], segidx_v, semA)
    pltpu.async_copy(gamma_hbm, g_v, semA)
    pltpu.async_copy(beta_hbm, b_v, semA)
    pltpu.async_copy(segtab_hbm, segbuf_v, semA)
    pltpu.make_async_copy(x_hbm.at[pl.ds(xrow, 2)], tokidx_v, semA).wait()
    pltpu.make_async_copy(seg_hbm.at[pl.ds(xrow, 2)], segidx_v, semA).wait()
    pltpu.make_async_copy(gamma_hbm, g_v, semA).wait()
    pltpu.make_async_copy(beta_hbm, b_v, semA).wait()
    pltpu.make_async_copy(segtab_hbm, segbuf_v, semA).wait()

    e0f = e0.at[0]
    e1f = e1.at[0]

    @pl.loop(0, 128)
    def _zfill(i):
        for j in range(8):
            e0f[i, pl.ds(j * _LANES, _LANES)] = zerov
            e1f[i, pl.ds(j * _LANES, _LANES)] = zerov

    @pl.loop(0, _NCHUNK)
    def _ofill(c):
        q0_ = (base + c * _C) * _J
        for j in range(8):
            oidx_v[c, pl.ds(j * _LANES, _LANES)] = q0_ + j * _LANES + lane

    @pl.loop(0, 8)
    def _pfill(p):
        posidx_v[pl.ds(p * _LANES, _LANES)] = \
            (l_base >> 4) + ((p * _LANES + lane) >> 3)

    def issue_adds(c, q_v, e_v, sem):
        r0 = c * _C
        tv = tokidx_v[c >> 3, pl.ds((c & 7) * _C, _C)]
        tq = ((tv >> 3) << 6) | (tv & 7)
        pv = l_base + r0 + lane
        pq = ((pv >> 3) << 6) | (pv & 7)
        for j in range(_J):
            q_v[pl.ds(j * _LANES, _LANES)] = tq + j * 8
            q_v[pl.ds(_QIDX + j * _LANES, _LANES)] = pq + j * 8
        pltpu.async_copy(tok_hbm.at[q_v.at[pl.ds(0, _QIDX)]],
                         e_v.at[0], sem, add=True)
        pltpu.async_copy(pos_hbm.at[q_v.at[pl.ds(_QIDX, _QIDX)]],
                         e_v.at[0], sem, add=True)

    def drain_adds(e_v, sem):
        for _ in range(2):
            pltpu.make_async_copy(tok_hbm.at[q0.at[pl.ds(0, _QIDX)]],
                                  e_v.at[0], sem).wait()

    def out_slice(c):
        return out_hbm.at[oidx_v.at[c]]

    def drain_out(c, o_v, sem):
        pltpu.make_async_copy(o_v, out_slice(c), sem).wait()

    issue_adds(0, q0, e0, semA)
    issue_adds(1, q1, e1, semB)

    @pl.loop(0, _NPAIR)
    def _pair(i):
        cA = 2 * i
        cB = 2 * i + 1

        svA = segidx_v[cA >> 3, pl.ds((cA & 7) * _C, _C)] << 3
        sgsA = [svA[t] for t in range(_C)]
        drain_adds(e0, semA)

        @pl.when(i > 0)
        def _():
            drain_out(cA, o0, semOA)

        _ln_chunk(e0, o0, g_v, b_v, segbuf_v, acc_v, stat_v, sgsA)
        pltpu.async_copy(o0, out_slice(cA), semOA)

        @pl.when(i < _NPAIR - 1)
        def _():
            issue_adds(cA + 2, q0, e0, semA)

        svB = segidx_v[cB >> 3, pl.ds((cB & 7) * _C, _C)] << 3
        sgsB = [svB[t] for t in range(_C)]
        drain_adds(e1, semB)

        @pl.when(i > 0)
        def _():
            drain_out(cB, o1, semOB)

        _ln_chunk(e1, o1, g_v, b_v, segbuf_v, acc_v, stat_v, sgsB)
        pltpu.async_copy(o1, out_slice(cB), semOB)

        @pl.when(i < _NPAIR - 1)
        def _():
            issue_adds(cB + 2, q1, e1, semB)

    drain_out(_NCHUNK - 2, o0, semOA)
    drain_out(_NCHUNK - 1, o1, semOB)


@jax.jit
def _emb(x2, seg2, tok_q, pos_q, seg_q, gamma, beta):
    mesh = plsc.VectorSubcoreMesh(core_axis_name="c", subcore_axis_name="s",
                                  num_cores=_NC, num_subcores=_NS)
    run = pl.kernel(
        _emb_body,
        out_type=jax.ShapeDtypeStruct((_N * _J, 128), jnp.float32),
        mesh=mesh,
        scratch_types=[
            pltpu.VMEM((2, 128), jnp.int32),
            pltpu.VMEM((2, 128), jnp.int32),
            pltpu.VMEM((2 * _QIDX,), jnp.int32),
            pltpu.VMEM((2 * _QIDX,), jnp.int32),
            pltpu.VMEM((_NCHUNK, 128), jnp.int32),
            pltpu.VMEM((128,), jnp.int32),
            pltpu.VMEM((1, _QIDX, 128), jnp.float32),
            pltpu.VMEM((1, _QIDX, 128), jnp.float32),
            pltpu.VMEM((_QIDX, 128), jnp.float32),
            pltpu.VMEM((_QIDX, 128), jnp.float32),
            pltpu.VMEM((16, 128), jnp.float32),
            pltpu.VMEM((_D,), jnp.float32),
            pltpu.VMEM((_D,), jnp.float32),
            pltpu.VMEM((2, 16 * _C), jnp.float32),
            pltpu.SMEM((2, _C), jnp.float32),
            pltpu.SemaphoreType.DMA,
            pltpu.SemaphoreType.DMA,
            pltpu.SemaphoreType.DMA,
            pltpu.SemaphoreType.DMA,
        ],
        compiler_params=pltpu.CompilerParams(use_tc_tiling_on_sc=False,
                                             needs_layout_passes=False),
    )
    return run(x2, seg2, tok_q, pos_q, seg_q, gamma, beta)


def _to_q(t):
    r = t.shape[0]
    return (t.reshape(r // 8, 8, 8, 128)
             .transpose(0, 2, 1, 3)
             .reshape(r * 8, 128))


def _from_q(o, B, L, D):
    return (o.reshape(B * L // 8, 8, 8, 128)
             .transpose(0, 2, 1, 3)
             .reshape(B, L, D))


def kernel(x, seg, mixup, shuffle_idx, l, clone_ids, mixup_layer, simple_pad,
           no_grad_clone, tok_table, pos_table, seg_table, gamma, beta):
    B, L = x.shape
    D = tok_table.shape[1]
    out = _emb(x.reshape(-1, 128), seg.reshape(-1, 128),
               _to_q(tok_table), _to_q(pos_table),
               seg_table.reshape(-1, 128), gamma, beta)
    return _from_q(out, B, L, D)
